# baseline (speedup 1.0000x reference)
import jax
import jax.numpy as jnp
from jax import lax
from jax.experimental import pallas as pl
from jax.experimental.pallas import tpu as pltpu
from jax.experimental.pallas import tpu_sc as plsc

NUM_CORES = 2
NUM_SUBCORES = 16
LANES = 16
NW = NUM_CORES * NUM_SUBCORES

NROWS = 1000000
BATCH = 16384
EMBED_DIM = 16
B_PER_W = BATCH // NW
GROUPS = B_PER_W // LANES
NVALS = B_PER_W * EMBED_DIM


def _dot_kernel(user_hbm, item_hbm, uflat_hbm, iflat_hbm, out_hbm,
                uidx_v, iidx_v, uoff_v, ioff_v, uval_v, ival_v, out_v,
                usem, isem):
    wid = lax.axis_index("s") * NUM_CORES + lax.axis_index("c")
    base = wid * B_PER_W

    pltpu.sync_copy(user_hbm.at[pl.ds(base, B_PER_W)], uidx_v)
    pltpu.sync_copy(item_hbm.at[pl.ds(base, B_PER_W)], iidx_v)

    def offsets(g, carry):
        uvec = uidx_v[pl.ds(g * LANES, LANES)]
        ivec = iidx_v[pl.ds(g * LANES, LANES)]
        for d in range(EMBED_DIM):
            uoff_v[pl.ds(d * B_PER_W + g * LANES, LANES)] = uvec + d * NROWS
            ioff_v[pl.ds(d * B_PER_W + g * LANES, LANES)] = ivec + d * NROWS
        return carry

    lax.fori_loop(0, GROUPS, offsets, 0)

    cu = pltpu.async_copy(uflat_hbm.at[uoff_v], uval_v, usem)
    ci = pltpu.async_copy(iflat_hbm.at[ioff_v], ival_v, isem)
    cu.wait()
    ci.wait()

    def compute(g, carry):
        sl = pl.ds(g * LANES, LANES)
        acc = jnp.zeros((LANES,), jnp.float32)
        for d in range(EMBED_DIM):
            dsl = pl.ds(d * B_PER_W + g * LANES, LANES)
            acc = acc + uval_v[dsl] * ival_v[dsl]
        out_v[sl] = acc
        return carry

    lax.fori_loop(0, GROUPS, compute, 0)

    pltpu.sync_copy(out_v, out_hbm.at[pl.ds(base, B_PER_W)])


@jax.jit
def kernel(user, item, user_emb, item_emb):
    mesh = plsc.VectorSubcoreMesh(
        core_axis_name="c", subcore_axis_name="s",
        num_cores=NUM_CORES, num_subcores=NUM_SUBCORES)
    run = pl.kernel(
        _dot_kernel,
        out_type=jax.ShapeDtypeStruct((BATCH,), jnp.float32),
        mesh=mesh,
        compiler_params=pltpu.CompilerParams(
            needs_layout_passes=False, use_tc_tiling_on_sc=False),
        scratch_types=[
            pltpu.VMEM((B_PER_W,), jnp.int32),
            pltpu.VMEM((B_PER_W,), jnp.int32),
            pltpu.VMEM((NVALS,), jnp.int32),
            pltpu.VMEM((NVALS,), jnp.int32),
            pltpu.VMEM((NVALS,), jnp.float32),
            pltpu.VMEM((NVALS,), jnp.float32),
            pltpu.VMEM((B_PER_W,), jnp.float32),
            pltpu.SemaphoreType.DMA,
            pltpu.SemaphoreType.DMA,
        ],
    )
    return run(user, item,
               user_emb.T.reshape(-1), item_emb.T.reshape(-1))


# speedup vs baseline: 1.0003x; 1.0003x over previous
import jax
import jax.numpy as jnp
from jax import lax
from jax.experimental import pallas as pl
from jax.experimental.pallas import tpu as pltpu
from jax.experimental.pallas import tpu_sc as plsc

NUM_CORES = 2
NUM_SUBCORES = 16
LANES = 16
NW = NUM_CORES * NUM_SUBCORES

NROWS = 1000000
BATCH = 16384
EMBED_DIM = 16
B_PER_W = BATCH // NW
GROUPS = B_PER_W // LANES


def _dot_kernel(user_hbm, item_hbm, uT_hbm, iT_hbm, out_hbm,
                uidx_v, iidx_v, uval_v, ival_v, out_v, usem, isem):
    wid = lax.axis_index("s") * NUM_CORES + lax.axis_index("c")
    base = wid * B_PER_W

    pltpu.sync_copy(user_hbm.at[pl.ds(base, B_PER_W)], uidx_v)
    pltpu.sync_copy(item_hbm.at[pl.ds(base, B_PER_W)], iidx_v)

    copies = []
    for d in range(EMBED_DIM):
        copies.append(pltpu.async_copy(
            uT_hbm.at[d].at[uidx_v],
            uval_v.at[pl.ds(d * B_PER_W, B_PER_W)], usem))
        copies.append(pltpu.async_copy(
            iT_hbm.at[d].at[iidx_v],
            ival_v.at[pl.ds(d * B_PER_W, B_PER_W)], isem))
    for c in copies:
        c.wait()

    def compute(g, carry):
        sl = pl.ds(g * LANES, LANES)
        acc = jnp.zeros((LANES,), jnp.float32)
        for d in range(EMBED_DIM):
            dsl = pl.ds(d * B_PER_W + g * LANES, LANES)
            acc = acc + uval_v[dsl] * ival_v[dsl]
        out_v[sl] = acc
        return carry

    lax.fori_loop(0, GROUPS, compute, 0)

    pltpu.sync_copy(out_v, out_hbm.at[pl.ds(base, B_PER_W)])


@jax.jit
def kernel(user, item, user_emb, item_emb):
    mesh = plsc.VectorSubcoreMesh(
        core_axis_name="c", subcore_axis_name="s",
        num_cores=NUM_CORES, num_subcores=NUM_SUBCORES)
    run = pl.kernel(
        _dot_kernel,
        out_type=jax.ShapeDtypeStruct((BATCH,), jnp.float32),
        mesh=mesh,
        compiler_params=pltpu.CompilerParams(
            needs_layout_passes=False, use_tc_tiling_on_sc=False),
        scratch_types=[
            pltpu.VMEM((B_PER_W,), jnp.int32),
            pltpu.VMEM((B_PER_W,), jnp.int32),
            pltpu.VMEM((B_PER_W * EMBED_DIM,), jnp.float32),
            pltpu.VMEM((B_PER_W * EMBED_DIM,), jnp.float32),
            pltpu.VMEM((B_PER_W,), jnp.float32),
            pltpu.SemaphoreType.DMA,
            pltpu.SemaphoreType.DMA,
        ],
    )
    return run(user, item, user_emb.T, item_emb.T)


# speedup vs baseline: 20.4836x; 20.4765x over previous
import jax
import jax.numpy as jnp
from jax import lax
from jax.experimental import pallas as pl
from jax.experimental.pallas import tpu as pltpu
from jax.experimental.pallas import tpu_sc as plsc

NUM_CORES = 2
NUM_SUBCORES = 16
LANES = 16
NW = NUM_CORES * NUM_SUBCORES

NROWS = 1000000
BATCH = 16384
EMBED_DIM = 16
B_PER_W = BATCH // NW
GROUPS = B_PER_W // LANES
NVALS = B_PER_W * EMBED_DIM

CHUNK = 65536
NCHUNKS = -(-NROWS // CHUNK)


def _detile_kernel(src_ref, dst_ref):
    for d in range(EMBED_DIM):
        dst_ref[pl.ds(d * CHUNK, CHUNK)] = src_ref[d, :]


def _linearize(tableT):
    return pl.pallas_call(
        _detile_kernel,
        grid=(NCHUNKS,),
        in_specs=[pl.BlockSpec((EMBED_DIM, CHUNK), lambda c: (0, c))],
        out_specs=pl.BlockSpec((EMBED_DIM * CHUNK,), lambda c: (c,)),
        out_shape=jax.ShapeDtypeStruct(
            (EMBED_DIM * CHUNK * NCHUNKS,), jnp.float32),
    )(tableT)


def _dot_kernel(user_hbm, item_hbm, uflat_hbm, iflat_hbm, out_hbm,
                uidx_v, iidx_v, uoff_v, ioff_v, uval_v, ival_v, out_v,
                usem, isem):
    wid = lax.axis_index("s") * NUM_CORES + lax.axis_index("c")
    base = wid * B_PER_W

    pltpu.sync_copy(user_hbm.at[pl.ds(base, B_PER_W)], uidx_v)
    pltpu.sync_copy(item_hbm.at[pl.ds(base, B_PER_W)], iidx_v)

    def offsets(g, carry):
        uvec = uidx_v[pl.ds(g * LANES, LANES)]
        ivec = iidx_v[pl.ds(g * LANES, LANES)]
        uq = uvec // CHUNK
        iq = ivec // CHUNK
        ubase = uq * (EMBED_DIM * CHUNK) + (uvec - uq * CHUNK)
        ibase = iq * (EMBED_DIM * CHUNK) + (ivec - iq * CHUNK)
        for d in range(EMBED_DIM):
            uoff_v[pl.ds(d * B_PER_W + g * LANES, LANES)] = ubase + d * CHUNK
            ioff_v[pl.ds(d * B_PER_W + g * LANES, LANES)] = ibase + d * CHUNK
        return carry

    lax.fori_loop(0, GROUPS, offsets, 0)

    cu = pltpu.async_copy(uflat_hbm.at[uoff_v], uval_v, usem)
    ci = pltpu.async_copy(iflat_hbm.at[ioff_v], ival_v, isem)
    cu.wait()
    ci.wait()

    def compute(g, carry):
        sl = pl.ds(g * LANES, LANES)
        acc = jnp.zeros((LANES,), jnp.float32)
        for d in range(EMBED_DIM):
            dsl = pl.ds(d * B_PER_W + g * LANES, LANES)
            acc = acc + uval_v[dsl] * ival_v[dsl]
        out_v[sl] = acc
        return carry

    lax.fori_loop(0, GROUPS, compute, 0)

    pltpu.sync_copy(out_v, out_hbm.at[pl.ds(base, B_PER_W)])


@jax.jit
def kernel(user, item, user_emb, item_emb):
    uflat = _linearize(user_emb.T)
    iflat = _linearize(item_emb.T)
    mesh = plsc.VectorSubcoreMesh(
        core_axis_name="c", subcore_axis_name="s",
        num_cores=NUM_CORES, num_subcores=NUM_SUBCORES)
    run = pl.kernel(
        _dot_kernel,
        out_type=jax.ShapeDtypeStruct((BATCH,), jnp.float32),
        mesh=mesh,
        compiler_params=pltpu.CompilerParams(
            needs_layout_passes=False, use_tc_tiling_on_sc=False),
        scratch_types=[
            pltpu.VMEM((B_PER_W,), jnp.int32),
            pltpu.VMEM((B_PER_W,), jnp.int32),
            pltpu.VMEM((NVALS,), jnp.int32),
            pltpu.VMEM((NVALS,), jnp.int32),
            pltpu.VMEM((NVALS,), jnp.float32),
            pltpu.VMEM((NVALS,), jnp.float32),
            pltpu.VMEM((B_PER_W,), jnp.float32),
            pltpu.SemaphoreType.DMA,
            pltpu.SemaphoreType.DMA,
        ],
    )
    return run(user, item, uflat, iflat)


# speedup vs baseline: 21.2636x; 1.0381x over previous
import jax
import jax.numpy as jnp
from jax import lax
from jax.experimental import pallas as pl
from jax.experimental.pallas import tpu as pltpu
from jax.experimental.pallas import tpu_sc as plsc

NUM_CORES = 2
NUM_SUBCORES = 16
LANES = 16
NW = NUM_CORES * NUM_SUBCORES

NROWS = 1000000
BATCH = 16384
EMBED_DIM = 16
B_PER_W = BATCH // NW
GROUPS = B_PER_W // LANES
NVALS = B_PER_W * EMBED_DIM

CHUNK = 65536
NCHUNKS = -(-NROWS // CHUNK)

_SC_PARAMS = dict(
    compiler_params=pltpu.CompilerParams(
        needs_layout_passes=False, use_tc_tiling_on_sc=False),
)


def _detile_kernel(src_ref, dst_ref):
    for d in range(EMBED_DIM):
        dst_ref[pl.ds(d * CHUNK, CHUNK)] = src_ref[d, :]


def _linearize(tableT):
    return pl.pallas_call(
        _detile_kernel,
        grid=(NCHUNKS,),
        in_specs=[pl.BlockSpec((EMBED_DIM, CHUNK), lambda c: (0, c))],
        out_specs=pl.BlockSpec((EMBED_DIM * CHUNK,), lambda c: (c,)),
        out_shape=jax.ShapeDtypeStruct(
            (EMBED_DIM * CHUNK * NCHUNKS,), jnp.float32),
    )(tableT)


def _flat_offsets(idx_ref, off_ref):
    def body(g, carry):
        vec = idx_ref[pl.ds(g * LANES, LANES)]
        q = vec // CHUNK
        base = q * (EMBED_DIM * CHUNK) + (vec - q * CHUNK)
        for d in range(EMBED_DIM):
            off_ref[pl.ds(d * B_PER_W + g * LANES, LANES)] = base + d * CHUNK
        return carry

    lax.fori_loop(0, GROUPS, body, 0)


def _gather_kernel(user_hbm, uflat_hbm, uvals_hbm,
                   uidx_v, uoff_v, uval_v, usem):
    wid = lax.axis_index("s") * NUM_CORES + lax.axis_index("c")
    base = wid * B_PER_W

    pltpu.sync_copy(user_hbm.at[pl.ds(base, B_PER_W)], uidx_v)
    _flat_offsets(uidx_v, uoff_v)
    pltpu.async_copy(uflat_hbm.at[uoff_v], uval_v, usem).wait()
    pltpu.sync_copy(uval_v, uvals_hbm.at[pl.ds(wid * NVALS, NVALS)])


def _dot_kernel(item_hbm, iflat_hbm, uvals_hbm, out_hbm,
                iidx_v, ioff_v, uval_v, ival_v, out_v, isem):
    wid = lax.axis_index("s") * NUM_CORES + lax.axis_index("c")
    base = wid * B_PER_W

    pltpu.sync_copy(item_hbm.at[pl.ds(base, B_PER_W)], iidx_v)
    _flat_offsets(iidx_v, ioff_v)
    ci = pltpu.async_copy(iflat_hbm.at[ioff_v], ival_v, isem)
    pltpu.sync_copy(uvals_hbm.at[pl.ds(wid * NVALS, NVALS)], uval_v)
    ci.wait()

    def compute(g, carry):
        sl = pl.ds(g * LANES, LANES)
        acc = jnp.zeros((LANES,), jnp.float32)
        for d in range(EMBED_DIM):
            dsl = pl.ds(d * B_PER_W + g * LANES, LANES)
            acc = acc + uval_v[dsl] * ival_v[dsl]
        out_v[sl] = acc
        return carry

    lax.fori_loop(0, GROUPS, compute, 0)

    pltpu.sync_copy(out_v, out_hbm.at[pl.ds(base, B_PER_W)])


@jax.jit
def kernel(user, item, user_emb, item_emb):
    mesh = plsc.VectorSubcoreMesh(
        core_axis_name="c", subcore_axis_name="s",
        num_cores=NUM_CORES, num_subcores=NUM_SUBCORES)

    uflat = _linearize(user_emb.T)
    gather_u = pl.kernel(
        _gather_kernel,
        out_type=jax.ShapeDtypeStruct((BATCH * EMBED_DIM,), jnp.float32),
        mesh=mesh,
        scratch_types=[
            pltpu.VMEM((B_PER_W,), jnp.int32),
            pltpu.VMEM((NVALS,), jnp.int32),
            pltpu.VMEM((NVALS,), jnp.float32),
            pltpu.SemaphoreType.DMA,
        ],
        **_SC_PARAMS,
    )
    uvals = gather_u(user, uflat)

    iflat = _linearize(item_emb.T)
    dot = pl.kernel(
        _dot_kernel,
        out_type=jax.ShapeDtypeStruct((BATCH,), jnp.float32),
        mesh=mesh,
        scratch_types=[
            pltpu.VMEM((B_PER_W,), jnp.int32),
            pltpu.VMEM((NVALS,), jnp.int32),
            pltpu.VMEM((NVALS,), jnp.float32),
            pltpu.VMEM((NVALS,), jnp.float32),
            pltpu.VMEM((B_PER_W,), jnp.float32),
            pltpu.SemaphoreType.DMA,
        ],
        **_SC_PARAMS,
    )
    return dot(item, iflat, uvals)


# speedup vs baseline: 21.7086x; 1.0209x over previous
import jax
import jax.numpy as jnp
from jax import lax
from jax.experimental import pallas as pl
from jax.experimental.pallas import tpu as pltpu
from jax.experimental.pallas import tpu_sc as plsc

NUM_CORES = 2
NUM_SUBCORES = 16
LANES = 16
NW = NUM_CORES * NUM_SUBCORES

NROWS = 1000000
BATCH = 16384
EMBED_DIM = 16
B_PER_W = BATCH // NW
GROUPS = B_PER_W // LANES
NVALS = B_PER_W * EMBED_DIM

CHUNK = 131072
NCHUNKS = -(-NROWS // CHUNK)

_SC_PARAMS = dict(
    compiler_params=pltpu.CompilerParams(
        needs_layout_passes=False, use_tc_tiling_on_sc=False),
)


def _detile_kernel(src_ref, dst_ref):
    for d in range(EMBED_DIM):
        dst_ref[pl.ds(d * CHUNK, CHUNK)] = src_ref[d, :]


def _linearize(tableT):
    return pl.pallas_call(
        _detile_kernel,
        grid=(NCHUNKS,),
        in_specs=[pl.BlockSpec((EMBED_DIM, CHUNK), lambda c: (0, c))],
        out_specs=pl.BlockSpec((EMBED_DIM * CHUNK,), lambda c: (c,)),
        out_shape=jax.ShapeDtypeStruct(
            (EMBED_DIM * CHUNK * NCHUNKS,), jnp.float32),
    )(tableT)


def _flat_offsets(idx_ref, off_ref):
    def body(g, carry):
        vec = idx_ref[pl.ds(g * LANES, LANES)]
        q = vec // CHUNK
        base = q * (EMBED_DIM * CHUNK) + (vec - q * CHUNK)
        for d in range(EMBED_DIM):
            off_ref[pl.ds(d * B_PER_W + g * LANES, LANES)] = base + d * CHUNK
        return carry

    lax.fori_loop(0, GROUPS, body, 0)


def _gather_kernel(user_hbm, uflat_hbm, uvals_hbm,
                   uidx_v, uoff_v, uval_v, usem):
    wid = lax.axis_index("s") * NUM_CORES + lax.axis_index("c")
    base = wid * B_PER_W

    pltpu.sync_copy(user_hbm.at[pl.ds(base, B_PER_W)], uidx_v)
    _flat_offsets(uidx_v, uoff_v)
    pltpu.async_copy(uflat_hbm.at[uoff_v], uval_v, usem).wait()
    pltpu.sync_copy(uval_v, uvals_hbm.at[pl.ds(wid * NVALS, NVALS)])


def _dot_kernel(item_hbm, iflat_hbm, uvals_hbm, out_hbm,
                iidx_v, ioff_v, uval_v, ival_v, out_v, isem):
    wid = lax.axis_index("s") * NUM_CORES + lax.axis_index("c")
    base = wid * B_PER_W

    pltpu.sync_copy(item_hbm.at[pl.ds(base, B_PER_W)], iidx_v)
    _flat_offsets(iidx_v, ioff_v)
    ci = pltpu.async_copy(iflat_hbm.at[ioff_v], ival_v, isem)
    pltpu.sync_copy(uvals_hbm.at[pl.ds(wid * NVALS, NVALS)], uval_v)
    ci.wait()

    def compute(g, carry):
        sl = pl.ds(g * LANES, LANES)
        acc = jnp.zeros((LANES,), jnp.float32)
        for d in range(EMBED_DIM):
            dsl = pl.ds(d * B_PER_W + g * LANES, LANES)
            acc = acc + uval_v[dsl] * ival_v[dsl]
        out_v[sl] = acc
        return carry

    lax.fori_loop(0, GROUPS, compute, 0)

    pltpu.sync_copy(out_v, out_hbm.at[pl.ds(base, B_PER_W)])


@jax.jit
def kernel(user, item, user_emb, item_emb):
    mesh = plsc.VectorSubcoreMesh(
        core_axis_name="c", subcore_axis_name="s",
        num_cores=NUM_CORES, num_subcores=NUM_SUBCORES)

    uflat = _linearize(user_emb.T)
    gather_u = pl.kernel(
        _gather_kernel,
        out_type=jax.ShapeDtypeStruct((BATCH * EMBED_DIM,), jnp.float32),
        mesh=mesh,
        scratch_types=[
            pltpu.VMEM((B_PER_W,), jnp.int32),
            pltpu.VMEM((NVALS,), jnp.int32),
            pltpu.VMEM((NVALS,), jnp.float32),
            pltpu.SemaphoreType.DMA,
        ],
        **_SC_PARAMS,
    )
    uvals = gather_u(user, uflat)

    iflat = _linearize(item_emb.T)
    dot = pl.kernel(
        _dot_kernel,
        out_type=jax.ShapeDtypeStruct((BATCH,), jnp.float32),
        mesh=mesh,
        scratch_types=[
            pltpu.VMEM((B_PER_W,), jnp.int32),
            pltpu.VMEM((NVALS,), jnp.int32),
            pltpu.VMEM((NVALS,), jnp.float32),
            pltpu.VMEM((NVALS,), jnp.float32),
            pltpu.VMEM((B_PER_W,), jnp.float32),
            pltpu.SemaphoreType.DMA,
        ],
        **_SC_PARAMS,
    )
    return dot(item, iflat, uvals)
